# Initial kernel scaffold; baseline (speedup 1.0000x reference)
import jax
import jax.numpy as jnp
from jax.experimental import pallas as pl


def kernel(desc_1, desc_2, inverse_T):
    raise NotImplementedError("write your pallas kernel here")


# speedup vs baseline: 2.0291x; 2.0291x over previous
import jax
import jax.numpy as jnp
from jax.experimental import pallas as pl
from jax.experimental.pallas import tpu as pltpu

_N, _M, _D = 4096, 4096, 64
_BR = 256


def _tile_e(t, d1, d2, cn):
    rn = jnp.sum(d1 * d1, axis=1, keepdims=True)
    ab = jax.lax.dot_general(
        d1, d2, (((1,), (1,)), ((), ())),
        preferred_element_type=jnp.float32)
    sq = jnp.maximum(rn + cn - 2.0 * ab, 0.0)
    return jnp.exp(-t * jnp.sqrt(sq + 1e-12))


def _matcher_kernel(t_ref, d1_ref, d2_ref, out_ref, cn_ref, rs_ref, cs_ref):
    p = pl.program_id(0)
    i = pl.program_id(1)
    t = t_ref[0, 0]
    d1 = d1_ref[...]
    d2 = d2_ref[...]

    @pl.when(jnp.logical_and(p == 0, i == 0))
    def _init():
        cn_ref[...] = jnp.sum(d2 * d2, axis=1)[None, :]
        cs_ref[...] = jnp.zeros_like(cs_ref)

    e = _tile_e(t, d1, d2, cn_ref[...])

    @pl.when(p == 0)
    def _stats():
        rs_ref[0, pl.ds(i * _BR, _BR)] = jnp.sum(e, axis=1)
        cs_ref[...] += jnp.sum(e, axis=0)[None, :]

    @pl.when(p == 1)
    def _emit():
        rinv = 1.0 / rs_ref[0, pl.ds(i * _BR, _BR)]
        cinv = 1.0 / cs_ref[0, :]
        out_ref[...] = (e * e) * rinv[:, None] * cinv[None, :]


def kernel(desc_1, desc_2, inverse_T):
    t = jnp.reshape(inverse_T.astype(jnp.float32), (1, 1))
    nb = _N // _BR
    return pl.pallas_call(
        _matcher_kernel,
        grid=(2, nb),
        in_specs=[
            pl.BlockSpec(memory_space=pltpu.SMEM),
            pl.BlockSpec((_BR, _D), lambda p, i: (i, 0)),
            pl.BlockSpec((_M, _D), lambda p, i: (0, 0)),
        ],
        out_specs=pl.BlockSpec((_BR, _M), lambda p, i: (p * i, 0)),
        out_shape=jax.ShapeDtypeStruct((_N, _M), jnp.float32),
        scratch_shapes=[
            pltpu.VMEM((1, _M), jnp.float32),
            pltpu.VMEM((1, _N), jnp.float32),
            pltpu.VMEM((1, _M), jnp.float32),
        ],
        compiler_params=pltpu.CompilerParams(
            dimension_semantics=("arbitrary", "arbitrary")),
    )(t, desc_1, desc_2)


# speedup vs baseline: 2.1698x; 1.0693x over previous
import jax
import jax.numpy as jnp
from jax.experimental import pallas as pl
from jax.experimental.pallas import tpu as pltpu

_N, _M, _D = 4096, 4096, 64
_BR = 256
_G = _BR // 8


def _matcher_kernel(t_ref, d1_ref, d2_ref, out_ref,
                    cn_ref, rs_ref, cs_ref, ci_ref):
    p = pl.program_id(0)
    i = pl.program_id(1)
    t = t_ref[0, 0]
    d1 = d1_ref[...]

    @pl.when(jnp.logical_and(p == 0, i == 0))
    def _init():
        d2 = d2_ref[...]
        cn = jnp.sum(d2 * d2, axis=1)[None, :]
        cn_ref[...] = jnp.broadcast_to(cn, (8, _M))
        cs_ref[...] = jnp.zeros_like(cs_ref)

    rn3 = jnp.sum(d1 * d1, axis=1, keepdims=True).reshape(_G, 8, 1)
    ab = jax.lax.dot_general(
        d1, d2_ref[...], (((1,), (1,)), ((), ())),
        preferred_element_type=jnp.float32)
    sq = jnp.maximum(rn3 + cn_ref[...][None] - 2.0 * ab.reshape(_G, 8, _M),
                     0.0)
    e3 = jnp.exp(-t * jnp.sqrt(sq + 1e-12))

    @pl.when(p == 0)
    def _stats():
        rs_ref[pl.ds(i * _G, _G), :] = jnp.sum(e3, axis=2)
        cs_ref[...] += jnp.sum(e3, axis=0)

    @pl.when(jnp.logical_and(p == 1, i == 0))
    def _colinv():
        tot = jnp.sum(cs_ref[...], axis=0, keepdims=True)
        ci_ref[...] = jnp.broadcast_to(1.0 / tot, (8, _M))

    @pl.when(p == 1)
    def _emit():
        rinv3 = (1.0 / rs_ref[pl.ds(i * _G, _G), :]).reshape(_G, 8, 1)
        out_ref[...] = ((e3 * e3) * rinv3 * ci_ref[...][None]
                        ).reshape(_BR, _M)


def kernel(desc_1, desc_2, inverse_T):
    t = jnp.reshape(inverse_T.astype(jnp.float32), (1, 1))
    nb = _N // _BR
    return pl.pallas_call(
        _matcher_kernel,
        grid=(2, nb),
        in_specs=[
            pl.BlockSpec(memory_space=pltpu.SMEM),
            pl.BlockSpec((_BR, _D), lambda p, i: (i, 0)),
            pl.BlockSpec((_M, _D), lambda p, i: (0, 0)),
        ],
        out_specs=pl.BlockSpec((_BR, _M), lambda p, i: (p * i, 0)),
        out_shape=jax.ShapeDtypeStruct((_N, _M), jnp.float32),
        scratch_shapes=[
            pltpu.VMEM((8, _M), jnp.float32),
            pltpu.VMEM((_N // 8, 8), jnp.float32),
            pltpu.VMEM((8, _M), jnp.float32),
            pltpu.VMEM((8, _M), jnp.float32),
        ],
        compiler_params=pltpu.CompilerParams(
            dimension_semantics=("arbitrary", "arbitrary")),
    )(t, desc_1, desc_2)


# speedup vs baseline: 3.0780x; 1.4185x over previous
import jax
import jax.numpy as jnp
from jax.experimental import pallas as pl
from jax.experimental.pallas import tpu as pltpu

_N, _M, _D = 4096, 4096, 64
_K = _D + 8
_BR = 256
_G = _BR // 8
_LOG2E = 1.4426950408889634


def _matcher_kernel(cd_ref, d1_ref, d2_ref, out_ref, rs_ref, cs_ref, ci_ref):
    p = pl.program_id(0)
    i = pl.program_id(1)
    cd = cd_ref[0, 0]

    @pl.when(jnp.logical_and(p == 0, i == 0))
    def _init():
        cs_ref[...] = jnp.zeros_like(cs_ref)

    sq = jax.lax.dot_general(
        d1_ref[...], d2_ref[...], (((1,), (1,)), ((), ())),
        preferred_element_type=jnp.float32)
    sq3 = jnp.maximum(sq.reshape(_G, 8, _M), 1e-12)
    arg = (cd * sq3) * jax.lax.rsqrt(sq3)

    @pl.when(p == 0)
    def _stats():
        e3 = jnp.exp2(arg)
        rs_ref[pl.ds(i * _G, _G), :] = jnp.sum(e3, axis=2)
        cs_ref[...] += jnp.sum(e3, axis=0)

    @pl.when(jnp.logical_and(p == 1, i == 0))
    def _colinv():
        tot = jnp.sum(cs_ref[...], axis=0, keepdims=True)
        ci_ref[...] = jnp.broadcast_to(1.0 / tot, (8, _M))

    @pl.when(p == 1)
    def _emit():
        f3 = jnp.exp2(arg + arg)
        rinv3 = (1.0 / rs_ref[pl.ds(i * _G, _G), :]).reshape(_G, 8, 1)
        out_ref[...] = (f3 * rinv3 * ci_ref[...][None]).reshape(_BR, _M)


def kernel(desc_1, desc_2, inverse_T):
    d1 = desc_1.astype(jnp.float32)
    d2 = desc_2.astype(jnp.float32)
    cd = jnp.reshape(-inverse_T.astype(jnp.float32) * _LOG2E, (1, 1))
    rn = jnp.sum(d1 * d1, axis=1, keepdims=True)
    cn = jnp.sum(d2 * d2, axis=1, keepdims=True) + 1e-12
    z1 = jnp.zeros((_N, _K - _D - 2), jnp.float32)
    z2 = jnp.zeros((_M, _K - _D - 2), jnp.float32)
    d1a = jnp.concatenate([d1, rn, jnp.ones_like(rn), z1], axis=1)
    d2a = jnp.concatenate([-2.0 * d2, jnp.ones_like(cn), cn, z2], axis=1)

    nb = _N // _BR
    return pl.pallas_call(
        _matcher_kernel,
        grid=(2, nb),
        in_specs=[
            pl.BlockSpec(memory_space=pltpu.SMEM),
            pl.BlockSpec((_BR, _K), lambda p, i: (i, 0)),
            pl.BlockSpec((_M, _K), lambda p, i: (0, 0)),
        ],
        out_specs=pl.BlockSpec((_BR, _M), lambda p, i: (p * i, 0)),
        out_shape=jax.ShapeDtypeStruct((_N, _M), jnp.float32),
        scratch_shapes=[
            pltpu.VMEM((_N // 8, 8), jnp.float32),
            pltpu.VMEM((8, _M), jnp.float32),
            pltpu.VMEM((8, _M), jnp.float32),
        ],
        compiler_params=pltpu.CompilerParams(
            dimension_semantics=("arbitrary", "arbitrary")),
    )(cd, d1a, d2a)


# speedup vs baseline: 3.3060x; 1.0741x over previous
import jax
import jax.numpy as jnp
from jax.experimental import pallas as pl
from jax.experimental.pallas import tpu as pltpu

_N, _M, _D = 4096, 4096, 64
_K = _D + 2
_BR = 256
_G = _BR // 8
_LOG2E = 1.4426950408889634


def _matcher_kernel(cd_ref, d1_ref, d2_ref, out_ref,
                    d1a_ref, d2a_ref, cs_ref, ci_ref):
    p = pl.program_id(0)
    i = pl.program_id(1)
    cd = cd_ref[0, 0]

    @pl.when(jnp.logical_and(p == 0, i == 0))
    def _init():
        d1 = d1_ref[...]
        d2 = d2_ref[...]
        d1a_ref[:, 0:_D] = d1
        d1a_ref[:, _D:_D + 1] = jnp.sum(d1 * d1, axis=1, keepdims=True)
        d1a_ref[:, _D + 1:_K] = jnp.ones((_N, 1), jnp.float32)
        d2a_ref[:, 0:_D] = -2.0 * d2
        d2a_ref[:, _D:_D + 1] = jnp.ones((_M, 1), jnp.float32)
        d2a_ref[:, _D + 1:_K] = (jnp.sum(d2 * d2, axis=1, keepdims=True)
                                 + 1e-12)
        cs_ref[...] = jnp.zeros_like(cs_ref)

    sq = jax.lax.dot_general(
        d1a_ref[pl.ds(i * _BR, _BR), :], d2a_ref[...],
        (((1,), (1,)), ((), ())),
        preferred_element_type=jnp.float32)
    sq3 = jnp.maximum(sq.reshape(_G, 8, _M), 1e-12)
    arg = (cd * sq3) * jax.lax.rsqrt(sq3)

    @pl.when(p == 0)
    def _stats():
        cs_ref[...] += jnp.sum(jnp.exp2(arg), axis=0)

    @pl.when(jnp.logical_and(p == 1, i == 0))
    def _colinv():
        tot = jnp.sum(cs_ref[...], axis=0, keepdims=True)
        ci_ref[...] = jnp.broadcast_to(1.0 / tot, (8, _M))

    @pl.when(p == 1)
    def _emit():
        e3 = jnp.exp2(arg)
        rinv3 = (1.0 / jnp.sum(e3, axis=2)).reshape(_G, 8, 1)
        out_ref[...] = ((e3 * e3) * rinv3 * ci_ref[...][None]
                        ).reshape(_BR, _M)


def kernel(desc_1, desc_2, inverse_T):
    cd = jnp.reshape(-inverse_T.astype(jnp.float32) * _LOG2E, (1, 1))
    nb = _N // _BR
    return pl.pallas_call(
        _matcher_kernel,
        grid=(2, nb),
        in_specs=[
            pl.BlockSpec(memory_space=pltpu.SMEM),
            pl.BlockSpec((_N, _D), lambda p, i: (0, 0)),
            pl.BlockSpec((_M, _D), lambda p, i: (0, 0)),
        ],
        out_specs=pl.BlockSpec((_BR, _M), lambda p, i: (p * i, 0)),
        out_shape=jax.ShapeDtypeStruct((_N, _M), jnp.float32),
        scratch_shapes=[
            pltpu.VMEM((_N, _K), jnp.float32),
            pltpu.VMEM((_M, _K), jnp.float32),
            pltpu.VMEM((8, _M), jnp.float32),
            pltpu.VMEM((8, _M), jnp.float32),
        ],
        compiler_params=pltpu.CompilerParams(
            dimension_semantics=("arbitrary", "arbitrary")),
    )(cd, desc_1.astype(jnp.float32), desc_2.astype(jnp.float32))
